# Initial kernel scaffold; baseline (speedup 1.0000x reference)
import functools

import jax
import jax.numpy as jnp
from jax import lax
from jax.experimental import pallas as pl
from jax.experimental.pallas import tpu as pltpu
from jax.experimental.pallas import tpu_sc as plsc

NC = 2
NS = 16
L = 16

_MESH = plsc.VectorSubcoreMesh(core_axis_name="c", subcore_axis_name="s")


def _deg_body(n, e, dst_hbm, parts_hbm, dstbuf, deg):
    cid = lax.axis_index("c")
    sid = lax.axis_index("s")
    wid = sid * NC + cid
    epw = e // (NC * NS)
    pltpu.sync_copy(dst_hbm.at[pl.ds(wid * epw, epw)], dstbuf)
    zeros = jnp.zeros((L,), jnp.float32)

    def zbody(i, c):
        deg[pl.ds(i * L, L)] = zeros
        return c

    lax.fori_loop(0, n // L, zbody, 0)
    ones = jnp.ones((L,), jnp.float32)

    def body(i, c):
        d16 = dstbuf[pl.ds(i * L, L)]
        plsc.addupdate_scatter(deg, [d16], ones)
        return c

    lax.fori_loop(0, epw // L, body, 0)
    pltpu.sync_copy(deg, parts_hbm.at[wid])


def _deg_call(dst, n, e):
    body = functools.partial(_deg_body, n, e)
    return pl.kernel(
        body,
        out_type=jax.ShapeDtypeStruct((NC * NS, n), jnp.float32),
        mesh=_MESH,
        scratch_types=[
            pltpu.VMEM((e // (NC * NS),), jnp.int32),
            pltpu.VMEM((n,), jnp.float32),
        ],
    )(dst)


_CH = 8000


def _scatter_body(n, e, gt_hbm, src_hbm, dst_hbm, parts_hbm,
                  hcol, acc, srcbuf, dstbuf):
    cid = lax.axis_index("c")
    sid = lax.axis_index("s")
    eps = e // NC
    base = cid * eps
    nch = eps // _CH
    pltpu.sync_copy(gt_hbm.at[sid], hcol)
    zeros = jnp.zeros((L,), jnp.float32)

    def zbody(i, c):
        acc[pl.ds(i * L, L)] = zeros
        return c

    lax.fori_loop(0, n // L, zbody, 0)

    def chunk_body(c, carry):
        ch = lax.rem(c + sid, nch)
        off = base + ch * _CH
        pltpu.sync_copy(src_hbm.at[pl.ds(off, _CH)], srcbuf)
        pltpu.sync_copy(dst_hbm.at[pl.ds(off, _CH)], dstbuf)

        def ib(i, cc):
            s16 = srcbuf[pl.ds(i * L, L)]
            d16 = dstbuf[pl.ds(i * L, L)]
            v = plsc.load_gather(hcol, [s16])
            plsc.addupdate_scatter(acc, [d16], v)
            return cc

        lax.fori_loop(0, _CH // L, ib, 0)
        return carry

    lax.fori_loop(0, nch, chunk_body, 0)
    pltpu.sync_copy(acc, parts_hbm.at[cid, sid])


def _scatter_call(gt, src, dst, n, e):
    body = functools.partial(_scatter_body, n, e)
    return pl.kernel(
        body,
        out_type=jax.ShapeDtypeStruct((NC, NS, n), jnp.float32),
        mesh=_MESH,
        scratch_types=[
            pltpu.VMEM((n,), jnp.float32),
            pltpu.VMEM((n,), jnp.float32),
            pltpu.VMEM((_CH,), jnp.int32),
            pltpu.VMEM((_CH,), jnp.int32),
        ],
    )(gt, src, dst)


_PREC = lax.Precision.HIGHEST


def _prep_body(x_ref, w1_ref, degp_ref, g_ref, dinv_ref):
    deg = jnp.sum(degp_ref[...], axis=0) + 1.0
    dinv = lax.rsqrt(deg)
    g = lax.dot_general(w1_ref[...], x_ref[...],
                        (((0,), (1,)), ((), ())),
                        precision=_PREC,
                        preferred_element_type=jnp.float32)
    g_ref[...] = g * dinv[None, :]
    dinv_ref[...] = dinv[None, :]


def _prep_call(x, w1, deg_parts, n, h):
    return pl.pallas_call(
        _prep_body,
        out_shape=[
            jax.ShapeDtypeStruct((h, n), jnp.float32),
            jax.ShapeDtypeStruct((1, n), jnp.float32),
        ],
    )(x, w1, deg_parts)


def _mid_body(parts_ref, g_ref, dinv_ref, w2_ref, b1_ref, out_ref):
    tot = parts_ref[0] + parts_ref[1] + g_ref[...]
    dinv = dinv_ref[...]
    hid = jnp.maximum(tot * dinv + b1_ref[...], 0.0)
    g2 = lax.dot_general(w2_ref[...], hid,
                         (((0,), (0,)), ((), ())),
                         precision=_PREC,
                         preferred_element_type=jnp.float32)
    out_ref[...] = g2 * dinv


def _mid_call(parts, g, dinv, w2, b1, n, h):
    return pl.pallas_call(
        _mid_body,
        out_shape=jax.ShapeDtypeStruct((h, n), jnp.float32),
    )(parts, g, dinv, w2, b1.reshape(h, 1))


def _fin_body(parts_ref, g_ref, dinv_ref, wf_ref, b2_ref, bf_ref, out_ref):
    tot = parts_ref[0] + parts_ref[1] + g_ref[...]
    hid = jnp.maximum(tot * dinv_ref[...] + b2_ref[...], 0.0)
    o = lax.dot_general(wf_ref[...], hid,
                        (((0,), (0,)), ((), ())),
                        precision=_PREC,
                        preferred_element_type=jnp.float32)
    out_ref[...] = jax.nn.sigmoid(o + bf_ref[...])


def _fin_call(parts, g, dinv, wf, b2, bf, n, h):
    return pl.pallas_call(
        _fin_body,
        out_shape=jax.ShapeDtypeStruct((1, n), jnp.float32),
    )(parts, g, dinv, wf, b2.reshape(h, 1), bf.reshape(1, 1))


def kernel(x, edge_index, edge_attr, batch, W1, b1, W2, b2, Wf, bf):
    n = x.shape[0]
    e = edge_index.shape[1]
    h = W1.shape[1]
    src = edge_index[0]
    dst = edge_index[1]

    deg_parts = _deg_call(dst, n, e)
    g1, dinv = _prep_call(x, W1, deg_parts, n, h)
    parts1 = _scatter_call(g1, src, dst, n, e)
    g2 = _mid_call(parts1, g1, dinv, W2, b1, n, h)
    parts2 = _scatter_call(g2, src, dst, n, e)
    out_t = _fin_call(parts2, g2, dinv, Wf, b2, bf, n, h)
    return out_t.reshape(n, 1)


# speedup vs baseline: 28.3308x; 28.3308x over previous
import functools

import jax
import jax.numpy as jnp
from jax import lax
from jax.experimental import pallas as pl
from jax.experimental.pallas import tpu as pltpu
from jax.experimental.pallas import tpu_sc as plsc

NC = 2
NS = 16
L = 16

_MESH = plsc.VectorSubcoreMesh(core_axis_name="c", subcore_axis_name="s")
_SC_PARAMS = pltpu.CompilerParams(needs_layout_passes=False)


def _deg_body(n, e, dst_hbm, parts_hbm, dstbuf, deg):
    cid = lax.axis_index("c")
    sid = lax.axis_index("s")
    wid = sid * NC + cid
    epw = e // (NC * NS)
    pltpu.sync_copy(dst_hbm.at[pl.ds(wid * epw, epw)], dstbuf)
    zeros = jnp.zeros((L,), jnp.float32)

    def zbody(i, c):
        deg[pl.ds(i * L, L)] = zeros
        return c

    lax.fori_loop(0, n // L, zbody, 0)
    ones = jnp.ones((L,), jnp.float32)

    def body(i, c):
        d16 = dstbuf[pl.ds(i * L, L)]
        plsc.addupdate_scatter(deg, [d16], ones)
        return c

    lax.fori_loop(0, epw // L, body, 0)
    pltpu.sync_copy(deg, parts_hbm.at[wid])


def _deg_call(dst, n, e):
    body = functools.partial(_deg_body, n, e)
    return pl.kernel(
        body,
        out_type=jax.ShapeDtypeStruct((NC * NS, n), jnp.float32),
        mesh=_MESH,
        scratch_types=[
            pltpu.VMEM((e // (NC * NS),), jnp.int32),
            pltpu.VMEM((n,), jnp.float32),
        ],
        compiler_params=_SC_PARAMS,
    )(dst)


_CH = 8000


def _scatter_body(n, e, gt_hbm, src_hbm, dst_hbm, parts_hbm,
                  hcol, acc, srcbuf, dstbuf):
    cid = lax.axis_index("c")
    sid = lax.axis_index("s")
    eps = e // NC
    base = cid * eps
    nch = eps // _CH
    pltpu.sync_copy(gt_hbm.at[sid], hcol)
    zeros = jnp.zeros((L,), jnp.float32)

    def zbody(i, c):
        acc[pl.ds(i * L, L)] = zeros
        return c

    lax.fori_loop(0, n // L, zbody, 0)

    def chunk_body(c, carry):
        ch = lax.rem(c + sid, nch)
        off = base + ch * _CH
        pltpu.sync_copy(src_hbm.at[pl.ds(off, _CH)], srcbuf)
        pltpu.sync_copy(dst_hbm.at[pl.ds(off, _CH)], dstbuf)

        def ib(i, cc):
            s16 = srcbuf[pl.ds(i * L, L)]
            d16 = dstbuf[pl.ds(i * L, L)]
            v = plsc.load_gather(hcol, [s16])
            plsc.addupdate_scatter(acc, [d16], v)
            return cc

        lax.fori_loop(0, _CH // L, ib, 0)
        return carry

    lax.fori_loop(0, nch, chunk_body, 0)
    pltpu.sync_copy(acc, parts_hbm.at[cid, sid])


def _scatter_call(gt, src, dst, n, e):
    body = functools.partial(_scatter_body, n, e)
    return pl.kernel(
        body,
        out_type=jax.ShapeDtypeStruct((NC, NS, n), jnp.float32),
        mesh=_MESH,
        scratch_types=[
            pltpu.VMEM((n,), jnp.float32),
            pltpu.VMEM((n,), jnp.float32),
            pltpu.VMEM((_CH,), jnp.int32),
            pltpu.VMEM((_CH,), jnp.int32),
        ],
        compiler_params=_SC_PARAMS,
    )(gt, src, dst)


_PREC = lax.Precision.HIGHEST


def _prep_body(x_ref, w1_ref, degp_ref, g_ref, dinv_ref):
    deg = jnp.sum(degp_ref[...], axis=0) + 1.0
    dinv = lax.rsqrt(deg)
    g = lax.dot_general(w1_ref[...], x_ref[...],
                        (((0,), (1,)), ((), ())),
                        precision=_PREC,
                        preferred_element_type=jnp.float32)
    g_ref[...] = g * dinv[None, :]
    dinv_ref[...] = dinv[None, :]


def _prep_call(x, w1, deg_parts, n, h):
    return pl.pallas_call(
        _prep_body,
        out_shape=[
            jax.ShapeDtypeStruct((h, n), jnp.float32),
            jax.ShapeDtypeStruct((1, n), jnp.float32),
        ],
    )(x, w1, deg_parts)


def _mid_body(parts_ref, g_ref, dinv_ref, w2_ref, b1_ref, out_ref):
    tot = parts_ref[0] + parts_ref[1] + g_ref[...]
    dinv = dinv_ref[...]
    hid = jnp.maximum(tot * dinv + b1_ref[...], 0.0)
    g2 = lax.dot_general(w2_ref[...], hid,
                         (((0,), (0,)), ((), ())),
                         precision=_PREC,
                         preferred_element_type=jnp.float32)
    out_ref[...] = g2 * dinv


def _mid_call(parts, g, dinv, w2, b1, n, h):
    return pl.pallas_call(
        _mid_body,
        out_shape=jax.ShapeDtypeStruct((h, n), jnp.float32),
    )(parts, g, dinv, w2, b1.reshape(h, 1))


def _fin_body(parts_ref, g_ref, dinv_ref, wf_ref, b2_ref, bf_ref, out_ref):
    tot = parts_ref[0] + parts_ref[1] + g_ref[...]
    hid = jnp.maximum(tot * dinv_ref[...] + b2_ref[...], 0.0)
    o = lax.dot_general(wf_ref[...], hid,
                        (((0,), (0,)), ((), ())),
                        precision=_PREC,
                        preferred_element_type=jnp.float32)
    out_ref[...] = jax.nn.sigmoid(o + bf_ref[...])


def _fin_call(parts, g, dinv, wf, b2, bf, n, h):
    return pl.pallas_call(
        _fin_body,
        out_shape=jax.ShapeDtypeStruct((1, n), jnp.float32),
    )(parts, g, dinv, wf, b2.reshape(h, 1), bf.reshape(1, 1))


def kernel(x, edge_index, edge_attr, batch, W1, b1, W2, b2, Wf, bf):
    n = x.shape[0]
    e = edge_index.shape[1]
    h = W1.shape[1]
    src = edge_index[0]
    dst = edge_index[1]

    deg_parts = _deg_call(dst, n, e)
    g1, dinv = _prep_call(x, W1, deg_parts, n, h)
    parts1 = _scatter_call(g1, src, dst, n, e)
    g2 = _mid_call(parts1, g1, dinv, W2, b1, n, h)
    parts2 = _scatter_call(g2, src, dst, n, e)
    out_t = _fin_call(parts2, g2, dinv, Wf, b2, bf, n, h)
    return out_t.reshape(n, 1)


# speedup vs baseline: 76.9765x; 2.7171x over previous
import functools

import jax
import jax.numpy as jnp
from jax import lax
from jax.experimental import pallas as pl
from jax.experimental.pallas import tpu as pltpu
from jax.experimental.pallas import tpu_sc as plsc

NC = 2
NS = 16
L = 16

_MESH = plsc.VectorSubcoreMesh(core_axis_name="c", subcore_axis_name="s")
_SC_PARAMS = pltpu.CompilerParams(needs_layout_passes=False)


def _deg_body(n, e, dst_hbm, parts_hbm, dstbuf, deg):
    cid = lax.axis_index("c")
    sid = lax.axis_index("s")
    wid = sid * NC + cid
    epw = e // (NC * NS)
    pltpu.sync_copy(dst_hbm.at[pl.ds(wid * epw, epw)], dstbuf)
    zeros = jnp.zeros((L,), jnp.float32)

    @plsc.parallel_loop(0, n // L, unroll=8)
    def _(i):
        deg[pl.ds(i * L, L)] = zeros

    ones = jnp.ones((L,), jnp.float32)

    @plsc.parallel_loop(0, epw // L, unroll=8)
    def _(i):
        d16 = dstbuf[pl.ds(i * L, L)]
        plsc.addupdate_scatter(deg, [d16], ones)

    pltpu.sync_copy(deg, parts_hbm.at[wid])


def _deg_call(dst, n, e):
    body = functools.partial(_deg_body, n, e)
    return pl.kernel(
        body,
        out_type=jax.ShapeDtypeStruct((NC * NS, n), jnp.float32),
        mesh=_MESH,
        scratch_types=[
            pltpu.VMEM((e // (NC * NS),), jnp.int32),
            pltpu.VMEM((n,), jnp.float32),
        ],
        compiler_params=_SC_PARAMS,
    )(dst)


_CH = 10000


def _scatter_body(n, e, gt_hbm, src_hbm, dst_hbm, parts_hbm,
                  hcol, acc, srcbuf0, srcbuf1, dstbuf0, dstbuf1,
                  s_sem0, s_sem1, d_sem0, d_sem1):
    cid = lax.axis_index("c")
    sid = lax.axis_index("s")
    eps = e // NC
    base = cid * eps
    nch = eps // _CH
    srcbufs = (srcbuf0, srcbuf1)
    dstbufs = (dstbuf0, dstbuf1)
    s_sems = (s_sem0, s_sem1)
    d_sems = (d_sem0, d_sem1)

    def start(c, slot):
        off = base + lax.rem(c + sid, nch) * _CH
        cs = pltpu.make_async_copy(src_hbm.at[pl.ds(off, _CH)],
                                   srcbufs[slot], s_sems[slot])
        cd = pltpu.make_async_copy(dst_hbm.at[pl.ds(off, _CH)],
                                   dstbufs[slot], d_sems[slot])
        cs.start()
        cd.start()
        return cs, cd

    pending = [None, None]
    pending[0] = start(0, 0)
    pltpu.sync_copy(gt_hbm.at[sid], hcol)
    zeros = jnp.zeros((L,), jnp.float32)

    @plsc.parallel_loop(0, n // L, unroll=8)
    def _(i):
        acc[pl.ds(i * L, L)] = zeros

    for c in range(nch):
        slot = c % 2
        if c + 1 < nch:
            pending[(c + 1) % 2] = start(c + 1, (c + 1) % 2)
        cs, cd = pending[slot]
        cs.wait()
        cd.wait()
        sb = srcbufs[slot]
        db = dstbufs[slot]

        @plsc.parallel_loop(0, _CH // L, unroll=5)
        def _(i):
            s16 = sb[pl.ds(i * L, L)]
            d16 = db[pl.ds(i * L, L)]
            v = plsc.load_gather(hcol, [s16])
            plsc.addupdate_scatter(acc, [d16], v)

    pltpu.sync_copy(acc, parts_hbm.at[cid, sid])


def _scatter_call(gt, src, dst, n, e):
    body = functools.partial(_scatter_body, n, e)
    return pl.kernel(
        body,
        out_type=jax.ShapeDtypeStruct((NC, NS, n), jnp.float32),
        mesh=_MESH,
        scratch_types=[
            pltpu.VMEM((n,), jnp.float32),
            pltpu.VMEM((n,), jnp.float32),
            pltpu.VMEM((_CH,), jnp.int32),
            pltpu.VMEM((_CH,), jnp.int32),
            pltpu.VMEM((_CH,), jnp.int32),
            pltpu.VMEM((_CH,), jnp.int32),
            pltpu.SemaphoreType.DMA,
            pltpu.SemaphoreType.DMA,
            pltpu.SemaphoreType.DMA,
            pltpu.SemaphoreType.DMA,
        ],
        compiler_params=_SC_PARAMS,
    )(gt, src, dst)


_PREC = lax.Precision.HIGHEST


def _prep_body(x_ref, w1_ref, degp_ref, g_ref, dinv_ref):
    deg = jnp.sum(degp_ref[...], axis=0) + 1.0
    dinv = lax.rsqrt(deg)
    g = lax.dot_general(w1_ref[...], x_ref[...],
                        (((0,), (1,)), ((), ())),
                        precision=_PREC,
                        preferred_element_type=jnp.float32)
    g_ref[...] = g * dinv[None, :]
    dinv_ref[...] = dinv[None, :]


def _prep_call(x, w1, deg_parts, n, h):
    return pl.pallas_call(
        _prep_body,
        out_shape=[
            jax.ShapeDtypeStruct((h, n), jnp.float32),
            jax.ShapeDtypeStruct((1, n), jnp.float32),
        ],
    )(x, w1, deg_parts)


def _mid_body(parts_ref, g_ref, dinv_ref, w2_ref, b1_ref, out_ref):
    tot = parts_ref[0] + parts_ref[1] + g_ref[...]
    dinv = dinv_ref[...]
    hid = jnp.maximum(tot * dinv + b1_ref[...], 0.0)
    g2 = lax.dot_general(w2_ref[...], hid,
                         (((0,), (0,)), ((), ())),
                         precision=_PREC,
                         preferred_element_type=jnp.float32)
    out_ref[...] = g2 * dinv


def _mid_call(parts, g, dinv, w2, b1, n, h):
    return pl.pallas_call(
        _mid_body,
        out_shape=jax.ShapeDtypeStruct((h, n), jnp.float32),
    )(parts, g, dinv, w2, b1.reshape(h, 1))


def _fin_body(parts_ref, g_ref, dinv_ref, wf_ref, b2_ref, bf_ref, out_ref):
    tot = parts_ref[0] + parts_ref[1] + g_ref[...]
    hid = jnp.maximum(tot * dinv_ref[...] + b2_ref[...], 0.0)
    o = lax.dot_general(wf_ref[...], hid,
                        (((0,), (0,)), ((), ())),
                        precision=_PREC,
                        preferred_element_type=jnp.float32)
    out_ref[...] = jax.nn.sigmoid(o + bf_ref[...])


def _fin_call(parts, g, dinv, wf, b2, bf, n, h):
    return pl.pallas_call(
        _fin_body,
        out_shape=jax.ShapeDtypeStruct((1, n), jnp.float32),
    )(parts, g, dinv, wf, b2.reshape(h, 1), bf.reshape(1, 1))


def kernel(x, edge_index, edge_attr, batch, W1, b1, W2, b2, Wf, bf):
    n = x.shape[0]
    e = edge_index.shape[1]
    h = W1.shape[1]
    src = edge_index[0]
    dst = edge_index[1]

    deg_parts = _deg_call(dst, n, e)
    g1, dinv = _prep_call(x, W1, deg_parts, n, h)
    parts1 = _scatter_call(g1, src, dst, n, e)
    g2 = _mid_call(parts1, g1, dinv, W2, b1, n, h)
    parts2 = _scatter_call(g2, src, dst, n, e)
    out_t = _fin_call(parts2, g2, dinv, Wf, b2, bf, n, h)
    return out_t.reshape(n, 1)
